# baseline (speedup 1.0000x reference)
import functools

import jax
import jax.numpy as jnp
from jax.experimental import pallas as pl
from jax.experimental.pallas import tpu as pltpu
from jax.experimental.pallas import tpu_sc as plsc

_B, _S, _D = 4096, 200, 64
_NC, _NS = 2, 16
_NW = _NC * _NS
_STEPS_PER_W = _B // _NW
_L = 16
_GWS = (128, 72)


def _sc_embed_add(x, mf, table):
    mesh = plsc.VectorSubcoreMesh(core_axis_name="core",
                                  subcore_axis_name="subcore")

    @functools.partial(
        pl.kernel,
        out_type=jax.ShapeDtypeStruct((_B, _S, _D), jnp.float32),
        mesh=mesh,
        compiler_params=pltpu.CompilerParams(use_tc_tiling_on_sc=False),
    )
    def k(x_hbm, i_hbm, t_hbm, o_hbm):
        def body(i_vmem, x_vmem, o_vmem):
            off = 0
            for gw in _GWS:
                pltpu.sync_copy(
                    t_hbm.at[i_vmem.at[0, pl.ds(off, gw)]],
                    o_vmem.at[0, pl.ds(off, gw)])
                off += gw

            @pl.loop(0, _S)
            def _(r):
                for c in range(_D // _L):
                    slc = (pl.ds(0, 1), pl.ds(r, 1), pl.ds(c * _L, _L))
                    o_vmem.at[slc][...] += x_vmem.at[slc][...]

        pltpu.emit_pipeline(
            body,
            grid=(_NC, _NS, _STEPS_PER_W),
            in_specs=[
                pl.BlockSpec(
                    (1, _S),
                    index_map=lambda i, j, k_: (
                        (i * _NS + j) * _STEPS_PER_W + k_, 0),
                ),
                pl.BlockSpec(
                    (1, _S, _D),
                    index_map=lambda i, j, k_: (
                        (i * _NS + j) * _STEPS_PER_W + k_, 0, 0),
                ),
            ],
            out_specs=[
                pl.BlockSpec(
                    (1, _S, _D),
                    index_map=lambda i, j, k_: (
                        (i * _NS + j) * _STEPS_PER_W + k_, 0, 0),
                ),
            ],
            core_axis_name=("core", "subcore"),
            dimension_semantics=(pltpu.PARALLEL, pltpu.PARALLEL,
                                 pltpu.ARBITRARY),
        )(i_hbm, x_hbm, o_hbm)

    return k(x, mf, table)


@jax.jit
def kernel(x, minutes, table):
    return _sc_embed_add(x, minutes.astype(jnp.int32), table)
